# baseline (speedup 1.0000x reference)
import functools

import jax
import jax.numpy as jnp
from jax import lax
from jax.experimental import pallas as pl
from jax.experimental.pallas import tpu as pltpu
from jax.experimental.pallas import tpu_sc as plsc

NUM_EMOTIONS = 1000
HIDDEN_DIM = 768
MAX_SEQ_LEN = 32
BATCH = 4096
D = MAX_SEQ_LEN * HIDDEN_DIM

MP = 128
MCHUNK = 32

NC = 2
NS = 16
NW = NC * NS
BPW = BATCH // NW
NBUF = 4
ROUNDS = BPW // NBUF


def _body(cond_hbm, masks_hbm, ids_hbm, ids2_hbm, out_h_hbm, out_m_hbm,
          idx1_v, idx_v, mrows_v, buf_v, gsems, wsems, msem):
    wid = lax.axis_index("s") * NC + lax.axis_index("c")
    base = wid * BPW

    pltpu.sync_copy(ids_hbm.at[pl.ds(base, BPW)], idx1_v)
    pltpu.sync_copy(ids2_hbm.at[pl.ds(base, BPW)], idx_v)

    def start_gather(g, b):
        pltpu.async_copy(cond_hbm.at[idx_v.at[g]], buf_v.at[b],
                         gsems.at[b])

    def wait_gather(g, b):
        pltpu.make_async_copy(cond_hbm.at[idx_v.at[g]],
                              buf_v.at[b], gsems.at[b]).wait()

    def start_write(g, b):
        pltpu.async_copy(buf_v.at[b], out_h_hbm.at[pl.ds(base + g, 1)],
                         wsems.at[b])

    def wait_write(g, b):
        pltpu.make_async_copy(buf_v.at[b], out_h_hbm.at[pl.ds(base + g, 1)],
                              wsems.at[b]).wait()

    for b in range(NBUF):
        start_gather(b, b)

    def round_body(o, _):
        for b in range(NBUF):
            g = o * NBUF + b
            wait_gather(g, b)
            start_write(g, b)
            wait_write(g, b)

            @pl.when(o < ROUNDS - 1)
            def _():
                start_gather(g + NBUF, b)
        return _

    lax.fori_loop(0, ROUNDS, round_body, None)

    for j in range(BPW // MCHUNK):
        pltpu.async_copy(
            masks_hbm.at[idx1_v.at[pl.ds(j * MCHUNK, MCHUNK)]],
            mrows_v, msem).wait()
        pltpu.sync_copy(mrows_v, out_m_hbm.at[pl.ds(base + j * MCHUNK,
                                                    MCHUNK)])


@jax.jit
def _launch(cond2d, masks, ids):
    mesh = plsc.VectorSubcoreMesh(core_axis_name="c", subcore_axis_name="s")
    f = pl.kernel(
        _body,
        out_type=(
            jax.ShapeDtypeStruct((BATCH, D), jnp.float32),
            jax.ShapeDtypeStruct((BATCH, MP), jnp.int32),
        ),
        mesh=mesh,
        scratch_types=[
            pltpu.VMEM((BPW,), jnp.int32),
            pltpu.VMEM((BPW, 1), jnp.int32),
            pltpu.VMEM((MCHUNK, MP), jnp.int32),
            pltpu.VMEM((NBUF, 1, D), jnp.float32),
            pltpu.SemaphoreType.DMA((NBUF,)),
            pltpu.SemaphoreType.DMA((NBUF,)),
            pltpu.SemaphoreType.DMA,
        ],
    )
    return f(cond2d, masks, ids, jnp.reshape(ids, (BATCH, 1)))


def kernel(conditioning, attention_masks, emotion_ids):
    cond2d = jnp.reshape(conditioning, (NUM_EMOTIONS, D))
    masks_pad = jnp.pad(attention_masks, ((0, 0), (0, MP - MAX_SEQ_LEN)))
    hidden2d, masks_out = _launch(cond2d, masks_pad, emotion_ids)
    return (jnp.reshape(hidden2d, (BATCH, MAX_SEQ_LEN, HIDDEN_DIM)),
            masks_out[:, :MAX_SEQ_LEN])


# speedup vs baseline: 2.1284x; 2.1284x over previous
import functools

import jax
import jax.numpy as jnp
from jax import lax
from jax.experimental import pallas as pl
from jax.experimental.pallas import tpu as pltpu
from jax.experimental.pallas import tpu_sc as plsc

NUM_EMOTIONS = 1000
HIDDEN_DIM = 768
MAX_SEQ_LEN = 32
BATCH = 4096
D = MAX_SEQ_LEN * HIDDEN_DIM

MP = 128
MCHUNK = 32

NC = 2
NS = 16
NW = NC * NS
BPW = BATCH // NW
NBUF = 4
ROUNDS = BPW // NBUF


def _body(cond_hbm, masks_hbm, ids_hbm, ids2_hbm, out_h_hbm, out_m_hbm,
          idx1_v, idx_v, mrows_v, buf_v, gsems, wsems, msem):
    wid = lax.axis_index("s") * NC + lax.axis_index("c")
    base = wid * BPW

    pltpu.sync_copy(ids_hbm.at[pl.ds(base, BPW)], idx1_v)
    pltpu.sync_copy(ids2_hbm.at[pl.ds(base, BPW)], idx_v)

    def start_gather(g, b):
        pltpu.async_copy(cond_hbm.at[idx_v.at[g]], buf_v.at[b],
                         gsems.at[b])

    def wait_gather(g, b):
        pltpu.make_async_copy(cond_hbm.at[idx_v.at[g]],
                              buf_v.at[b], gsems.at[b]).wait()

    def start_write(g, b):
        pltpu.async_copy(buf_v.at[b], out_h_hbm.at[pl.ds(base + g, 1)],
                         wsems.at[b])

    def wait_write(g, b):
        pltpu.make_async_copy(buf_v.at[b], out_h_hbm.at[pl.ds(base + g, 1)],
                              wsems.at[b]).wait()

    for b in range(NBUF):
        start_gather(b, b)

    def round_body(o, _):
        for b in range(NBUF):
            g = o * NBUF + b
            wait_gather(g, b)
            start_write(g, b)
            wait_write(g, b)

            @pl.when(o < ROUNDS - 1)
            def _():
                start_gather(g + NBUF, b)
        return _

    lax.fori_loop(0, ROUNDS, round_body, None)

    for j in range(BPW // MCHUNK):
        pltpu.async_copy(
            masks_hbm.at[idx1_v.at[pl.ds(j * MCHUNK, MCHUNK)]],
            mrows_v, msem).wait()
        pltpu.sync_copy(mrows_v, out_m_hbm.at[pl.ds(base + j * MCHUNK,
                                                    MCHUNK)])


@jax.jit
def _launch(cond2d, masks, ids):
    mesh = plsc.VectorSubcoreMesh(core_axis_name="c", subcore_axis_name="s")
    f = pl.kernel(
        _body,
        out_type=(
            jax.ShapeDtypeStruct((BATCH, MAX_SEQ_LEN, HIDDEN_DIM),
                                 jnp.float32),
            jax.ShapeDtypeStruct((BATCH, MP), jnp.int32),
        ),
        mesh=mesh,
        scratch_types=[
            pltpu.VMEM((BPW,), jnp.int32),
            pltpu.VMEM((BPW, 1), jnp.int32),
            pltpu.VMEM((MCHUNK, MP), jnp.int32),
            pltpu.VMEM((NBUF, 1, MAX_SEQ_LEN, HIDDEN_DIM), jnp.float32),
            pltpu.SemaphoreType.DMA((NBUF,)),
            pltpu.SemaphoreType.DMA((NBUF,)),
            pltpu.SemaphoreType.DMA,
        ],
    )
    return f(cond2d, masks, ids, jnp.reshape(ids, (BATCH, 1)))


def kernel(conditioning, attention_masks, emotion_ids):
    masks_pad = jnp.pad(attention_masks, ((0, 0), (0, MP - MAX_SEQ_LEN)))
    hidden, masks_out = _launch(conditioning, masks_pad, emotion_ids)
    return (hidden, masks_out[:, :MAX_SEQ_LEN])
